# baseline (speedup 1.0000x reference)
import functools

import jax
import jax.numpy as jnp
from jax import lax
from jax.experimental import pallas as pl
from jax.experimental.pallas import tpu as pltpu
from jax.experimental.pallas import tpu_sc as plsc

_BATCH = 16384
_EMBED = 64

_info = plsc.get_sparse_core_info()
_NC = _info.num_cores
_NS = _info.num_subcores
_NW = _NC * _NS
_BPW = _BATCH // _NW

_mesh = plsc.VectorSubcoreMesh(core_axis_name="c", subcore_axis_name="s")


@functools.partial(
    pl.kernel,
    mesh=_mesh,
    compiler_params=pltpu.CompilerParams(use_tc_tiling_on_sc=False),
    out_type=[
        jax.ShapeDtypeStruct((_BATCH, _EMBED), jnp.float32),
        jax.ShapeDtypeStruct((_BATCH, _EMBED), jnp.float32),
        jax.ShapeDtypeStruct((_BATCH, _EMBED), jnp.float32),
    ],
    scratch_types=[
        pltpu.VMEM((_BPW,), jnp.int32),
        pltpu.VMEM((_BPW,), jnp.int32),
        pltpu.VMEM((_BPW,), jnp.int32),
        pltpu.VMEM((_BPW, _EMBED), jnp.float32),
        pltpu.VMEM((_BPW, _EMBED), jnp.float32),
        pltpu.VMEM((_BPW, _EMBED), jnp.float32),
        pltpu.SemaphoreType.DMA,
        pltpu.SemaphoreType.DMA,
        pltpu.SemaphoreType.DMA,
    ],
)
def _sbpr_gather(user_hbm, pos_hbm, neg_hbm, eu_hbm, ei_hbm,
                 out_u, out_p, out_n,
                 idx_u, idx_p, idx_n,
                 rows_u, rows_p, rows_n,
                 sem_u, sem_p, sem_n):
    wid = lax.axis_index("s") * _NC + lax.axis_index("c")
    base = wid * _BPW
    pltpu.sync_copy(user_hbm.at[pl.ds(base, _BPW)], idx_u)
    pltpu.sync_copy(pos_hbm.at[pl.ds(base, _BPW)], idx_p)
    pltpu.sync_copy(neg_hbm.at[pl.ds(base, _BPW)], idx_n)
    cu = pltpu.async_copy(eu_hbm.at[idx_u], rows_u, sem_u)
    cp = pltpu.async_copy(ei_hbm.at[idx_p], rows_p, sem_p)
    cn = pltpu.async_copy(ei_hbm.at[idx_n], rows_n, sem_n)
    cu.wait()
    pltpu.sync_copy(rows_u, out_u.at[pl.ds(base, _BPW)])
    cp.wait()
    pltpu.sync_copy(rows_p, out_p.at[pl.ds(base, _BPW)])
    cn.wait()
    pltpu.sync_copy(rows_n, out_n.at[pl.ds(base, _BPW)])


@jax.jit
def kernel(user, pos_item, neg_item, embed_user, embed_item):
    return tuple(_sbpr_gather(user, pos_item, neg_item, embed_user, embed_item))
